# baseline (speedup 1.0000x reference)
import jax
import jax.numpy as jnp
from jax.experimental import pallas as pl

N = 10000
E = 160000
H = 2
C = 256
HC = H * C
NG = 8


def _gat_layer(z, edge_index, p):
    n = z.shape[0]
    src, dst = edge_index[0], edge_index[1]
    h = (z @ p['W']).reshape(n, H, C)
    a_s = jnp.sum(h * p['a_src'][None, :, :], axis=-1)
    a_d = jnp.sum(h * p['a_dst'][None, :, :], axis=-1)
    e = jax.nn.leaky_relu(a_s[src] + a_d[dst], 0.2)
    m = jax.ops.segment_max(e, dst, num_segments=n)
    m = jnp.where(jnp.isneginf(m), 0.0, m)
    ex = jnp.exp(e - m[dst])
    denom = jax.ops.segment_sum(ex, dst, num_segments=n)
    alpha = ex / (denom[dst] + 1e-16)
    out = jax.ops.segment_sum(h[src] * alpha[:, :, None], dst, num_segments=n)
    out = out.reshape(n, HC) + p['b']
    zz = out @ p['W_lin'] + p['b_lin']
    zz = jax.nn.leaky_relu(zz, 0.2)
    mu = jnp.mean(zz, axis=0)
    var = jnp.var(zz, axis=0)
    zz = (zz - mu) / jnp.sqrt(var + 1e-5) * p['gamma'] + p['beta']
    return zz


def _mlp_kernel(g_ref, w1_ref, b1_ref, w2_ref, b2_ref, w3_ref, b3_ref, o_ref):
    h1 = jax.nn.leaky_relu(g_ref[...] @ w1_ref[...] + b1_ref[...], 0.2)
    h2 = jax.nn.leaky_relu(h1 @ w2_ref[...] + b2_ref[...], 0.2)
    o_ref[...] = h2 @ w3_ref[...] + b3_ref[...]


def kernel(x, edge_index, edge_attr, batch, params):
    z = x
    for lp in params['layers']:
        z = _gat_layer(z, edge_index, lp)
    counts = jax.ops.segment_sum(jnp.ones((z.shape[0],), z.dtype), batch, num_segments=NG)
    sums = jax.ops.segment_sum(z, batch, num_segments=NG)
    g = sums / jnp.maximum(counts, 1.0)[:, None]
    f = params['fcn']
    out = pl.pallas_call(
        _mlp_kernel,
        out_shape=jax.ShapeDtypeStruct((NG, 2), jnp.float32),
    )(g, f['w1'], f['b1'][None, :], f['w2'], f['b2'][None, :], f['w3'], f['b3'][None, :])
    return out


# speedup vs baseline: 23.7070x; 23.7070x over previous
import functools

import jax
import jax.numpy as jnp
from jax import lax
from jax.experimental import pallas as pl
from jax.experimental.pallas import tpu as pltpu
import jax.experimental.pallas.tpu_sc as plsc

N = 10000
E = 160000
H = 2
C = 256
HC = H * C
NG = 8

K = 128
RR = E // K
ROWS = 1280
RPT = ROWS // 16
SR = 16
CW = 128
NCH = HC // CW
PP = NCH // 2
ND = 10240
BN = 1000
GRID = N // BN


def _mm_body(z_ref, stats_ref, gamma_ref, beta_ref, w_ref, av_ref, h_ref, acat_ref):
    mu = stats_ref[0:1, :] / N
    var = stats_ref[1:2, :] / N - mu * mu
    inv = lax.rsqrt(var + 1e-5) * gamma_ref[...]
    z = (z_ref[...] - mu) * inv + beta_ref[...]
    h = jnp.dot(z, w_ref[...], preferred_element_type=jnp.float32)
    for c in range(NCH):
        h_ref[c] = h[:, c * CW:(c + 1) * CW]
    acat_ref[...] = lax.dot_general(
        h, av_ref[...], (((1,), (1,)), ((), ())),
        preferred_element_type=jnp.float32)


def _run_mm(z, stats, gamma, beta, w, av):
    din = z.shape[1]
    return pl.pallas_call(
        _mm_body,
        grid=(GRID,),
        in_specs=[
            pl.BlockSpec((BN, din), lambda i: (i, 0)),
            pl.BlockSpec((2, din), lambda i: (0, 0)),
            pl.BlockSpec((1, din), lambda i: (0, 0)),
            pl.BlockSpec((1, din), lambda i: (0, 0)),
            pl.BlockSpec((din, HC), lambda i: (0, 0)),
            pl.BlockSpec((4, HC), lambda i: (0, 0)),
        ],
        out_specs=[
            pl.BlockSpec((NCH, BN, CW), lambda i: (0, i, 0)),
            pl.BlockSpec((BN, 4), lambda i: (i, 0)),
        ],
        out_shape=[
            jax.ShapeDtypeStruct((NCH, N, CW), jnp.float32),
            jax.ShapeDtypeStruct((N, 4), jnp.float32),
        ],
    )(z, stats, gamma, beta, w, av)


def _zero_acc(zrow, acc, sid):
    nb = pl.multiple_of(sid * 624, 8)
    pltpu.sync_copy(zrow, acc.at[pl.ds(nb, 624)])

    @pl.when(sid == 15)
    def _():
        pltpu.sync_copy(zrow.at[pl.ds(0, 16)], acc.at[pl.ds(9984, 16)])


def _copy_out(acc, out_cm, chunk, sid):
    nb = pl.multiple_of(sid * 624, 8)
    pltpu.sync_copy(acc.at[pl.ds(nb, 624)], out_cm.at[chunk, pl.ds(nb, 624)])

    @pl.when(sid == 15)
    def _():
        pltpu.sync_copy(acc.at[pl.ds(9984, 16)],
                        out_cm.at[chunk, pl.ds(9984, 16)])


def _sc_body(tabp, srcr, dstr, hflat, zrow, zvec,
             out_cm, denom,
             tab_sd, zbuf, src_loc, dst_loc, w_row, idx_loc, didx_loc,
             rows_buf, acc, dacc, gsem0, gsem1, ssem0, ssem1, dsem):
    head = lax.axis_index("c")
    sid = lax.axis_index("s")
    rbase = sid * RPT
    hbase = pl.multiple_of(head * N, 8)

    @pl.when(sid == 0)
    def _():
        pltpu.sync_copy(zvec, zbuf)
        for k in range(ND // 1280):
            pltpu.sync_copy(zbuf, dacc.at[pl.ds(k * 1280, 1280)])

    pltpu.sync_copy(tabp.at[pl.ds(hbase, N)], tab_sd)

    _zero_acc(zrow, acc, sid)
    plsc.subcore_barrier()

    gsems = (gsem0, gsem1)
    ssems = (ssem0, ssem1)

    def sweep(p, with_denom):
        cb = (head * PP + p) * N

        def build_idx(i, b):
            for j in range(8):
                sl = pl.ds(j * 16, 16)
                idx_loc[b, sl] = src_loc[i, sl] + cb
                didx_loc[b, sl] = dst_loc[i, sl]

        def start_gather(b):
            pltpu.async_copy(hflat.at[idx_loc.at[b]], rows_buf.at[b], gsems[b])

        def wait_gather(b):
            pltpu.make_async_copy(hflat.at[idx_loc.at[b]],
                                  rows_buf.at[b], gsems[b]).wait()

        def wait_scatter(b):
            pltpu.make_async_copy(rows_buf.at[b], acc.at[didx_loc.at[b]],
                                  ssems[b]).wait()

        def chunk_body(cc, ccarry):
            coff = pl.multiple_of(rbase + cc * SR, 8)
            pltpu.sync_copy(srcr.at[pl.ds(coff, SR)], src_loc)
            pltpu.sync_copy(dstr.at[pl.ds(coff, SR)], dst_loc)
            build_idx(0, 0)
            start_gather(0)

            def pair_body(rr, carry):
                for half in range(2):
                    i = 2 * rr + half
                    b = half
                    nb = 1 - half
                    live = (rbase + cc * SR + i) < RR
                    mf = lax.convert_element_type(live, jnp.float32)
                    for j in range(8):
                        sl = pl.ds(j * 16, 16)
                        gs = plsc.load_gather(tab_sd, [src_loc[i, sl]])
                        gd = plsc.load_gather(tab_sd, [dst_loc[i, sl]])
                        a_s, _ = plsc.unpack(
                            plsc.bitcast(gs, jnp.bfloat16),
                            format=plsc.PackFormat.INTERLEAVED)
                        _, a_d = plsc.unpack(
                            plsc.bitcast(gd, jnp.bfloat16),
                            format=plsc.PackFormat.INTERLEAVED)
                        e = a_s + a_d
                        e = jnp.where(e > 0, e, e * 0.2)
                        w_row[sl] = jnp.exp(e) * mf
                    if with_denom:
                        ddesc = pltpu.async_copy(w_row, dacc.at[dst_loc.at[i]],
                                                 dsem, add=True)
                    wait_gather(b)

                    def g_body(g, c2, b=b):
                        wv = w_row[pl.ds(g * 16, 16)]
                        for lane in range(16):
                            ws = wv[lane]
                            ke = g * 16 + lane
                            for j in range(CW // 16):
                                sl = pl.ds(j * 16, 16)
                                rows_buf[b, ke, sl] = rows_buf[b, ke, sl] * ws
                        return c2

                    lax.fori_loop(0, 8, g_body, 0)
                    if half == 0:
                        @pl.when(rr > 0)
                        def _():
                            wait_scatter(nb)

                        build_idx(i + 1, nb)
                        start_gather(nb)
                    else:
                        wait_scatter(nb)

                        @pl.when(rr + 1 < SR // 2)
                        def _():
                            build_idx(i + 1, nb)
                            start_gather(nb)
                    pltpu.async_copy(rows_buf.at[b], acc.at[didx_loc.at[b]],
                                     ssems[b], add=True)
                    if with_denom:
                        ddesc.wait()
                return carry

            lax.fori_loop(0, SR // 2, pair_body, 0)
            wait_scatter(1)
            return ccarry

        lax.fori_loop(0, RPT // SR, chunk_body, 0)

    sweep(0, True)
    plsc.subcore_barrier()
    _copy_out(acc, out_cm, head * PP, sid)
    _zero_acc(zrow, acc, sid)
    plsc.subcore_barrier()
    sweep(1, False)
    plsc.subcore_barrier()
    _copy_out(acc, out_cm, head * PP + 1, sid)

    @pl.when(sid == 0)
    def _():
        for k in range(ND // 1280):
            pltpu.sync_copy(dacc.at[pl.ds(k * 1280, 1280)], zbuf)
            pltpu.sync_copy(
                zbuf,
                denom.at[pl.ds(pl.multiple_of(head * ND + k * 1280, 8), 1280)])


def _run_sc(tabp, srcp, dstp, hflat, zrow, zvec):
    return pl.kernel(
        _sc_body,
        out_type=[
            jax.ShapeDtypeStruct((NCH, N, CW), jnp.float32),
            jax.ShapeDtypeStruct((2 * ND,), jnp.float32),
        ],
        mesh=plsc.VectorSubcoreMesh(core_axis_name="c", subcore_axis_name="s"),
        compiler_params=pltpu.CompilerParams(needs_layout_passes=False),
        scratch_types=[
            pltpu.VMEM((N,), jnp.int32),
            pltpu.VMEM((1280,), jnp.float32),
            pltpu.VMEM((SR, K), jnp.int32),
            pltpu.VMEM((SR, K), jnp.int32),
            pltpu.VMEM((K,), jnp.float32),
            pltpu.VMEM((2, K), jnp.int32),
            pltpu.VMEM((2, K), jnp.int32),
            pltpu.VMEM((2, K, CW), jnp.float32),
            pltpu.VMEM_SHARED((N, CW), jnp.float32),
            pltpu.VMEM_SHARED((ND,), jnp.float32),
        ] + [pltpu.SemaphoreType.DMA] * 5,
    )(tabp, srcp, dstp, hflat, zrow, zvec)


def _lin_body(ocm_ref, den_ref, b_ref, wl_ref, bl_ref, zz_ref, stats_ref):
    i = pl.program_id(0)
    o = jnp.concatenate([ocm_ref[c] for c in range(NCH)],
                        axis=1).astype(jnp.float32)
    s0 = 1.0 / (den_ref[:, 0:1] + 1e-16)
    s1 = 1.0 / (den_ref[:, 1:2] + 1e-16)
    o = o * jnp.concatenate([jnp.broadcast_to(s0, (BN, C)),
                             jnp.broadcast_to(s1, (BN, C))], axis=1)
    o = o + b_ref[...]
    zz = jnp.dot(o, wl_ref[...], preferred_element_type=jnp.float32) + bl_ref[...]
    zz = jnp.where(zz > 0, zz, zz * 0.2)
    zz_ref[...] = zz

    @pl.when(i == 0)
    def _():
        stats_ref[...] = jnp.zeros_like(stats_ref)

    stats_ref[0:1, :] += jnp.sum(zz, axis=0, keepdims=True)
    stats_ref[1:2, :] += jnp.sum(zz * zz, axis=0, keepdims=True)


def _run_lin(ocm, den, b, wl, bl):
    return pl.pallas_call(
        _lin_body,
        grid=(GRID,),
        in_specs=[
            pl.BlockSpec((NCH, BN, CW), lambda i: (0, i, 0)),
            pl.BlockSpec((BN, 2), lambda i: (i, 0)),
            pl.BlockSpec((1, HC), lambda i: (0, 0)),
            pl.BlockSpec((HC, C), lambda i: (0, 0)),
            pl.BlockSpec((1, C), lambda i: (0, 0)),
        ],
        out_specs=[
            pl.BlockSpec((BN, C), lambda i: (i, 0)),
            pl.BlockSpec((2, C), lambda i: (0, 0)),
        ],
        out_shape=[
            jax.ShapeDtypeStruct((N, C), jnp.float32),
            jax.ShapeDtypeStruct((2, C), jnp.float32),
        ],
    )(ocm, den, b, wl, bl)


def _pool_body(zz_ref, stats_ref, gamma_ref, beta_ref, bt_ref,
               w1_ref, b1_ref, w2_ref, b2_ref, w3_ref, b3_ref,
               out_ref, g_acc, c_acc):
    i = pl.program_id(0)

    @pl.when(i == 0)
    def _():
        g_acc[...] = jnp.zeros_like(g_acc)
        c_acc[...] = jnp.zeros_like(c_acc)

    mu = stats_ref[0:1, :] / N
    var = stats_ref[1:2, :] / N - mu * mu
    inv = lax.rsqrt(var + 1e-5) * gamma_ref[...]
    z = (zz_ref[...] - mu) * inv + beta_ref[...]
    oh = (bt_ref[0] == lax.broadcasted_iota(jnp.int32, (NG, BN), 0))
    oh = oh.astype(jnp.float32)
    g_acc[...] += jnp.dot(oh, z, preferred_element_type=jnp.float32)
    c_acc[...] = c_acc[...] + jnp.sum(oh, axis=1, keepdims=True)

    @pl.when(i == pl.num_programs(0) - 1)
    def _():
        g = g_acc[...] / jnp.maximum(c_acc[...][:, 0:1], 1.0)
        h1 = jnp.dot(g, w1_ref[...], preferred_element_type=jnp.float32) + b1_ref[...]
        h1 = jnp.where(h1 > 0, h1, h1 * 0.2)
        h2 = jnp.dot(h1, w2_ref[...], preferred_element_type=jnp.float32) + b2_ref[...]
        h2 = jnp.where(h2 > 0, h2, h2 * 0.2)
        out_ref[...] = jnp.dot(h2, w3_ref[...], preferred_element_type=jnp.float32) + b3_ref[...]


def _run_pool(zz, stats, gamma, beta, bt, f):
    return pl.pallas_call(
        _pool_body,
        grid=(GRID,),
        in_specs=[
            pl.BlockSpec((BN, C), lambda i: (i, 0)),
            pl.BlockSpec((2, C), lambda i: (0, 0)),
            pl.BlockSpec((1, C), lambda i: (0, 0)),
            pl.BlockSpec((1, C), lambda i: (0, 0)),
            pl.BlockSpec((1, 1, BN), lambda i: (i, 0, 0)),
            pl.BlockSpec((C, C), lambda i: (0, 0)),
            pl.BlockSpec((1, C), lambda i: (0, 0)),
            pl.BlockSpec((C, 32), lambda i: (0, 0)),
            pl.BlockSpec((1, 32), lambda i: (0, 0)),
            pl.BlockSpec((32, 2), lambda i: (0, 0)),
            pl.BlockSpec((1, 2), lambda i: (0, 0)),
        ],
        out_specs=pl.BlockSpec((NG, 2), lambda i: (0, 0)),
        out_shape=jax.ShapeDtypeStruct((NG, 2), jnp.float32),
        scratch_shapes=[
            pltpu.VMEM((NG, C), jnp.float32),
            pltpu.VMEM((NG, 128), jnp.float32),
        ],
    )(zz, stats, gamma, beta, bt, f['w1'], f['b1'][None, :], f['w2'],
      f['b2'][None, :], f['w3'], f['b3'][None, :])


def kernel(x, edge_index, edge_attr, batch, params):
    src = edge_index[0]
    dst = edge_index[1]
    pad = jnp.zeros((ROWS * K - E,), src.dtype)
    srcp = jnp.concatenate([src, pad]).reshape(ROWS, K).astype(jnp.int32)
    dstp = jnp.concatenate([dst, pad]).reshape(ROWS, K).astype(jnp.int32)
    zrow = jnp.zeros((624, CW), jnp.float32)
    zvec = jnp.zeros((1280,), jnp.float32)
    bt = batch.astype(jnp.int32).reshape(GRID, 1, BN)

    zz = x
    din0 = x.shape[1]
    stats = jnp.stack([jnp.zeros((din0,), jnp.float32),
                       jnp.full((din0,), float(N), jnp.float32)])
    gamma = jnp.ones((1, din0), jnp.float32)
    beta = jnp.zeros((1, din0), jnp.float32)

    azc = jnp.zeros((C,), jnp.float32)
    for lp in params['layers']:
        av = jnp.stack([
            jnp.concatenate([lp['a_src'][0], azc]),
            jnp.concatenate([azc, lp['a_src'][1]]),
            jnp.concatenate([lp['a_dst'][0], azc]),
            jnp.concatenate([azc, lp['a_dst'][1]]),
        ])
        h_cm, acat_nt = _run_mm(zz, stats, gamma, beta, lp['W'], av)
        tabs = []
        for h in range(H):
            pair = jnp.stack([acat_nt[:, h], acat_nt[:, 2 + h]],
                             axis=-1).astype(jnp.bfloat16)
            tabs.append(lax.bitcast_convert_type(pair, jnp.int32))
        tabp = jnp.concatenate(tabs)
        hflat = h_cm.reshape(NCH * N, CW)
        out_cm, denom = _run_sc(tabp, srcp, dstp, hflat, zrow, zvec)
        den = denom.reshape(2, ND)[:, :N].T
        zz, stats = _run_lin(out_cm, den, lp['b'][None, :],
                             lp['W_lin'], lp['b_lin'][None, :])
        gamma = lp['gamma'][None, :]
        beta = lp['beta'][None, :]

    return _run_pool(zz, stats, gamma, beta, bt, params['fcn'])


# speedup vs baseline: 27.3635x; 1.1542x over previous
import functools

import jax
import jax.numpy as jnp
from jax import lax
from jax.experimental import pallas as pl
from jax.experimental.pallas import tpu as pltpu
import jax.experimental.pallas.tpu_sc as plsc

N = 10000
E = 160000
H = 2
C = 256
HC = H * C
NG = 8

K = 128
RR = E // K
ROWS = 1280
RPT = ROWS // 16
CW = 128
NCH = HC // CW
PP = NCH // 2
BN = 1000
GRID = N // BN


def _mm_body(z_ref, stats_ref, gamma_ref, beta_ref, w_ref, av_ref, h_ref, acat_ref):
    mu = stats_ref[0:1, :] / N
    var = stats_ref[1:2, :] / N - mu * mu
    inv = lax.rsqrt(var + 1e-5) * gamma_ref[...]
    z = (z_ref[...] - mu) * inv + beta_ref[...]
    h = jnp.dot(z, w_ref[...], preferred_element_type=jnp.float32)
    for c in range(NCH):
        h_ref[c] = h[:, c * CW:(c + 1) * CW]
    acat_ref[...] = lax.dot_general(
        h, av_ref[...], (((1,), (1,)), ((), ())),
        preferred_element_type=jnp.float32)


def _run_mm(z, stats, gamma, beta, w, av):
    din = z.shape[1]
    return pl.pallas_call(
        _mm_body,
        grid=(GRID,),
        in_specs=[
            pl.BlockSpec((BN, din), lambda i: (i, 0)),
            pl.BlockSpec((2, din), lambda i: (0, 0)),
            pl.BlockSpec((1, din), lambda i: (0, 0)),
            pl.BlockSpec((1, din), lambda i: (0, 0)),
            pl.BlockSpec((din, HC), lambda i: (0, 0)),
            pl.BlockSpec((4, HC), lambda i: (0, 0)),
        ],
        out_specs=[
            pl.BlockSpec((NCH, BN, CW), lambda i: (0, i, 0)),
            pl.BlockSpec((BN, 4), lambda i: (i, 0)),
        ],
        out_shape=[
            jax.ShapeDtypeStruct((NCH, N, CW), jnp.float32),
            jax.ShapeDtypeStruct((N, 4), jnp.float32),
        ],
    )(z, stats, gamma, beta, w, av)


def _zero_acc(zrow, acc, sid):
    nb = pl.multiple_of(sid * 624, 8)
    pltpu.sync_copy(zrow, acc.at[pl.ds(nb, 624)])

    @pl.when(sid == 15)
    def _():
        pltpu.sync_copy(zrow.at[pl.ds(0, 16)], acc.at[pl.ds(9984, 16)])


def _copy_out(acc, out_cm, chunk, sid):
    nb = pl.multiple_of(sid * 624, 8)
    pltpu.sync_copy(acc.at[pl.ds(nb, 624)], out_cm.at[chunk, pl.ds(nb, 624)])

    @pl.when(sid == 15)
    def _():
        pltpu.sync_copy(acc.at[pl.ds(9984, 16)],
                        out_cm.at[chunk, pl.ds(9984, 16)])


def _sc_body(acat, srcr, dstr, hflat, zrow, zvec,
             out_cm, denom,
             tab_s, tab_d, src_loc, dst_loc, w_row, idx_loc, didx_loc,
             rows_buf, acc, dacc,
             gsem0, gsem1, gsem2, gsem3, ssem0, ssem1, ssem2, ssem3, dsem):
    head = lax.axis_index("c")
    sid = lax.axis_index("s")
    rbase = sid * RPT
    hbase = pl.multiple_of(head * N, 8)

    @pl.when(sid == 0)
    def _():
        pltpu.sync_copy(zvec, tab_s)
        pltpu.sync_copy(tab_s, dacc)

    pltpu.sync_copy(acat.at[pl.ds(hbase, N)], tab_s)
    pltpu.sync_copy(acat.at[pl.ds(pl.multiple_of(2 * N + hbase, 8), N)], tab_d)

    _zero_acc(zrow, acc, sid)
    plsc.subcore_barrier()

    gsems = (gsem0, gsem1, gsem2, gsem3)
    ssems = (ssem0, ssem1, ssem2, ssem3)

    def sweep(p, with_denom):
        cb = (head * PP + p) * N
        SB = 32
        HR = RPT // 2

        def build_idx(i, q, b):
            for j in range(2):
                sl16 = pl.ds(q * SB + j * 16, 16)
                dsl = pl.ds(j * 16, 16)
                idx_loc[b, dsl] = src_loc[i, sl16] + cb
                didx_loc[b, dsl] = dst_loc[i, sl16]

        def start_gather(b):
            pltpu.async_copy(hflat.at[idx_loc.at[b]], rows_buf.at[b], gsems[b])

        def wait_gather(b):
            pltpu.make_async_copy(hflat.at[idx_loc.at[b]],
                                  rows_buf.at[b], gsems[b]).wait()

        def wait_scatter(b):
            pltpu.make_async_copy(rows_buf.at[b], acc.at[didx_loc.at[b]],
                                  ssems[b]).wait()

        for hh in range(2):
            pltpu.sync_copy(srcr.at[pl.ds(rbase + hh * HR, HR)], src_loc)
            pltpu.sync_copy(dstr.at[pl.ds(rbase + hh * HR, HR)], dst_loc)
            build_idx(0, 0, 0)
            start_gather(0)
            build_idx(0, 1, 1)
            start_gather(1)

            def row_body(i, carry):
                live = (rbase + hh * HR + i) < RR
                mf = lax.convert_element_type(live, jnp.float32)
                for j in range(8):
                    sl = pl.ds(j * 16, 16)
                    sidx = src_loc[i, sl]
                    didx = dst_loc[i, sl]
                    e = (plsc.load_gather(tab_s, [sidx])
                         + plsc.load_gather(tab_d, [didx]))
                    e = jnp.where(e > 0, e, e * 0.2)
                    w_row[sl] = jnp.exp(e) * mf
                if with_denom:
                    ddesc = pltpu.async_copy(w_row, dacc.at[dst_loc.at[i]],
                                             dsem, add=True)
                for q in range(4):
                    b = q
                    gb = (q + 2) % 4
                    wait_gather(b)
                    wv = w_row[pl.ds(q * SB, 16)]
                    wv2 = w_row[pl.ds(q * SB + 16, 16)]
                    for half, wvx in ((0, wv), (1, wv2)):
                        for lane in range(16):
                            ws = wvx[lane]
                            ke = half * 16 + lane
                            for j in range(CW // 16):
                                sl = pl.ds(j * 16, 16)
                                rows_buf[b, ke, sl] = rows_buf[b, ke, sl] * ws
                    if q < 2:
                        @pl.when(i > 0)
                        def _():
                            wait_scatter(gb)
                    else:
                        wait_scatter(gb)
                    if q < 2:
                        build_idx(i, q + 2, gb)
                        start_gather(gb)
                    else:
                        @pl.when(i + 1 < HR)
                        def _():
                            build_idx(i + 1, q - 2, gb)
                            start_gather(gb)
                    pltpu.async_copy(rows_buf.at[b], acc.at[didx_loc.at[b]],
                                     ssems[b], add=True)
                if with_denom:
                    ddesc.wait()
                return carry

            lax.fori_loop(0, HR, row_body, 0)
            wait_scatter(2)
            wait_scatter(3)

    sweep(0, True)
    plsc.subcore_barrier()
    _copy_out(acc, out_cm, head * PP, sid)

    @pl.when(sid == 0)
    def _():
        pltpu.sync_copy(dacc, tab_s)
        pltpu.sync_copy(tab_s, denom.at[pl.ds(hbase, N)])

    _zero_acc(zrow, acc, sid)
    plsc.subcore_barrier()
    sweep(1, False)
    plsc.subcore_barrier()
    _copy_out(acc, out_cm, head * PP + 1, sid)


def _run_sc(acat, srcp, dstp, hflat, zrow, zvec):
    return pl.kernel(
        _sc_body,
        out_type=[
            jax.ShapeDtypeStruct((NCH, N, CW), jnp.float32),
            jax.ShapeDtypeStruct((2 * N,), jnp.float32),
        ],
        mesh=plsc.VectorSubcoreMesh(core_axis_name="c", subcore_axis_name="s"),
        compiler_params=pltpu.CompilerParams(needs_layout_passes=False),
        scratch_types=[
            pltpu.VMEM((N,), jnp.float32),
            pltpu.VMEM((N,), jnp.float32),
            pltpu.VMEM((RPT // 2, K), jnp.int32),
            pltpu.VMEM((RPT // 2, K), jnp.int32),
            pltpu.VMEM((K,), jnp.float32),
            pltpu.VMEM((4, 32), jnp.int32),
            pltpu.VMEM((4, 32), jnp.int32),
            pltpu.VMEM((4, 32, CW), jnp.float32),
            pltpu.VMEM_SHARED((N, CW), jnp.float32),
            pltpu.VMEM_SHARED((N,), jnp.float32),
        ] + [pltpu.SemaphoreType.DMA] * 9,
    )(acat, srcp, dstp, hflat, zrow, zvec)


def _lin_body(ocm_ref, den_ref, b_ref, wl_ref, bl_ref, zz_ref, stats_ref):
    i = pl.program_id(0)
    o = jnp.concatenate([ocm_ref[c] for c in range(NCH)],
                        axis=1).astype(jnp.float32)
    s0 = 1.0 / (den_ref[:, 0:1] + 1e-16)
    s1 = 1.0 / (den_ref[:, 1:2] + 1e-16)
    o = o * jnp.concatenate([jnp.broadcast_to(s0, (BN, C)),
                             jnp.broadcast_to(s1, (BN, C))], axis=1)
    o = o + b_ref[...]
    zz = jnp.dot(o, wl_ref[...], preferred_element_type=jnp.float32) + bl_ref[...]
    zz = jnp.where(zz > 0, zz, zz * 0.2)
    zz_ref[...] = zz

    @pl.when(i == 0)
    def _():
        stats_ref[...] = jnp.zeros_like(stats_ref)

    stats_ref[0:1, :] += jnp.sum(zz, axis=0, keepdims=True)
    stats_ref[1:2, :] += jnp.sum(zz * zz, axis=0, keepdims=True)


def _run_lin(ocm, den, b, wl, bl):
    return pl.pallas_call(
        _lin_body,
        grid=(GRID,),
        in_specs=[
            pl.BlockSpec((NCH, BN, CW), lambda i: (0, i, 0)),
            pl.BlockSpec((BN, 2), lambda i: (i, 0)),
            pl.BlockSpec((1, HC), lambda i: (0, 0)),
            pl.BlockSpec((HC, C), lambda i: (0, 0)),
            pl.BlockSpec((1, C), lambda i: (0, 0)),
        ],
        out_specs=[
            pl.BlockSpec((BN, C), lambda i: (i, 0)),
            pl.BlockSpec((2, C), lambda i: (0, 0)),
        ],
        out_shape=[
            jax.ShapeDtypeStruct((N, C), jnp.float32),
            jax.ShapeDtypeStruct((2, C), jnp.float32),
        ],
    )(ocm, den, b, wl, bl)


def _pool_body(zz_ref, stats_ref, gamma_ref, beta_ref, bt_ref,
               w1_ref, b1_ref, w2_ref, b2_ref, w3_ref, b3_ref,
               out_ref, g_acc, c_acc):
    i = pl.program_id(0)

    @pl.when(i == 0)
    def _():
        g_acc[...] = jnp.zeros_like(g_acc)
        c_acc[...] = jnp.zeros_like(c_acc)

    mu = stats_ref[0:1, :] / N
    var = stats_ref[1:2, :] / N - mu * mu
    inv = lax.rsqrt(var + 1e-5) * gamma_ref[...]
    z = (zz_ref[...] - mu) * inv + beta_ref[...]
    oh = (bt_ref[0] == lax.broadcasted_iota(jnp.int32, (NG, BN), 0))
    oh = oh.astype(jnp.float32)
    g_acc[...] += jnp.dot(oh, z, preferred_element_type=jnp.float32)
    c_acc[...] = c_acc[...] + jnp.sum(oh, axis=1, keepdims=True)

    @pl.when(i == pl.num_programs(0) - 1)
    def _():
        g = g_acc[...] / jnp.maximum(c_acc[...][:, 0:1], 1.0)
        h1 = jnp.dot(g, w1_ref[...], preferred_element_type=jnp.float32) + b1_ref[...]
        h1 = jnp.where(h1 > 0, h1, h1 * 0.2)
        h2 = jnp.dot(h1, w2_ref[...], preferred_element_type=jnp.float32) + b2_ref[...]
        h2 = jnp.where(h2 > 0, h2, h2 * 0.2)
        out_ref[...] = jnp.dot(h2, w3_ref[...], preferred_element_type=jnp.float32) + b3_ref[...]


def _run_pool(zz, stats, gamma, beta, bt, f):
    return pl.pallas_call(
        _pool_body,
        grid=(GRID,),
        in_specs=[
            pl.BlockSpec((BN, C), lambda i: (i, 0)),
            pl.BlockSpec((2, C), lambda i: (0, 0)),
            pl.BlockSpec((1, C), lambda i: (0, 0)),
            pl.BlockSpec((1, C), lambda i: (0, 0)),
            pl.BlockSpec((1, 1, BN), lambda i: (i, 0, 0)),
            pl.BlockSpec((C, C), lambda i: (0, 0)),
            pl.BlockSpec((1, C), lambda i: (0, 0)),
            pl.BlockSpec((C, 32), lambda i: (0, 0)),
            pl.BlockSpec((1, 32), lambda i: (0, 0)),
            pl.BlockSpec((32, 2), lambda i: (0, 0)),
            pl.BlockSpec((1, 2), lambda i: (0, 0)),
        ],
        out_specs=pl.BlockSpec((NG, 2), lambda i: (0, 0)),
        out_shape=jax.ShapeDtypeStruct((NG, 2), jnp.float32),
        scratch_shapes=[
            pltpu.VMEM((NG, C), jnp.float32),
            pltpu.VMEM((NG, 128), jnp.float32),
        ],
    )(zz, stats, gamma, beta, bt, f['w1'], f['b1'][None, :], f['w2'],
      f['b2'][None, :], f['w3'], f['b3'][None, :])


def kernel(x, edge_index, edge_attr, batch, params):
    src = edge_index[0]
    dst = edge_index[1]
    pad = jnp.zeros((ROWS * K - E,), src.dtype)
    srcp = jnp.concatenate([src, pad]).reshape(ROWS, K).astype(jnp.int32)
    dstp = jnp.concatenate([dst, pad]).reshape(ROWS, K).astype(jnp.int32)
    zrow = jnp.zeros((624, CW), jnp.float32)
    zvec = jnp.zeros((N,), jnp.float32)
    bt = batch.astype(jnp.int32).reshape(GRID, 1, BN)

    zz = x
    din0 = x.shape[1]
    stats = jnp.stack([jnp.zeros((din0,), jnp.float32),
                       jnp.full((din0,), float(N), jnp.float32)])
    gamma = jnp.ones((1, din0), jnp.float32)
    beta = jnp.zeros((1, din0), jnp.float32)

    azc = jnp.zeros((C,), jnp.float32)
    for lp in params['layers']:
        av = jnp.stack([
            jnp.concatenate([lp['a_src'][0], azc]),
            jnp.concatenate([azc, lp['a_src'][1]]),
            jnp.concatenate([lp['a_dst'][0], azc]),
            jnp.concatenate([azc, lp['a_dst'][1]]),
        ])
        h_cm, acat_nt = _run_mm(zz, stats, gamma, beta, lp['W'], av)
        acat = acat_nt.T.reshape(4 * N)
        hflat = h_cm.reshape(NCH * N, CW)
        out_cm, denom = _run_sc(acat, srcp, dstp, hflat, zrow, zvec)
        zz, stats = _run_lin(out_cm, denom.reshape(2, N).T, lp['b'][None, :],
                             lp['W_lin'], lp['b_lin'][None, :])
        gamma = lp['gamma'][None, :]
        beta = lp['beta'][None, :]

    return _run_pool(zz, stats, gamma, beta, bt, params['fcn'])
